# Initial kernel scaffold; baseline (speedup 1.0000x reference)
import functools

import jax
import jax.numpy as jnp
from jax import lax
from jax.experimental import pallas as pl
from jax.experimental.pallas import tpu as pltpu
from jax.experimental.pallas import tpu_sc as plsc

_DIM = 256
_NE = 8192
_NTOK = 8192
_BLK = 512


def _argmin_body(zb_ref, cb_ref, idx_ref):
    zb = zb_ref[...]
    cb = cb_ref[...]
    z2 = jnp.sum(zb * zb, axis=1, keepdims=True)
    e2 = jnp.sum(cb * cb, axis=1)
    prod = lax.dot_general(zb, cb, (((1,), (1,)), ((), ())),
                           preferred_element_type=jnp.float32)
    d = (z2 + e2) - 2.0 * prod
    m = jnp.min(d, axis=1, keepdims=True)
    ids = lax.broadcasted_iota(jnp.int32, d.shape, 1)
    idx_ref[...] = jnp.min(jnp.where(d == m, ids, jnp.int32(_NE)), axis=1)


def _build_argmin(interpret: bool = False):
    return pl.pallas_call(
        _argmin_body,
        grid=(_NTOK // _BLK,),
        in_specs=[
            pl.BlockSpec((_BLK, _DIM), lambda i: (i, 0)),
            pl.BlockSpec((_NE, _DIM), lambda i: (0, 0)),
        ],
        out_specs=pl.BlockSpec((_BLK,), lambda i: (i,)),
        out_shape=jax.ShapeDtypeStruct((_NTOK,), jnp.int32),
        compiler_params=pltpu.CompilerParams(
            dimension_semantics=("arbitrary",)),
        interpret=interpret,
    )


_tc_argmin = _build_argmin()

_SC = plsc.get_sparse_core_info()
_NW = _SC.num_cores * _SC.num_subcores
_BPW = _NTOK // _NW
_CH = 128

_sc_mesh = plsc.VectorSubcoreMesh(core_axis_name="c", subcore_axis_name="s")


@functools.partial(
    pl.kernel,
    out_type=jax.ShapeDtypeStruct((_NTOK, _DIM), jnp.float32),
    mesh=_sc_mesh,
    scratch_types=[
        pltpu.VMEM((_CH,), jnp.int32),
        pltpu.VMEM((_CH,), jnp.int32),
        pltpu.VMEM((_CH, _DIM), jnp.float32),
        pltpu.VMEM((_CH, _DIM), jnp.float32),
        pltpu.SemaphoreType.DMA,
    ],
)
def _sc_gather(cb_hbm, idx_hbm, out_hbm, idx_a, idx_b, rows_a, rows_b, sem):
    wid = lax.axis_index("s") * _SC.num_cores + lax.axis_index("c")
    base = wid * _BPW
    pltpu.sync_copy(idx_hbm.at[pl.ds(base, _CH)], idx_a)
    pltpu.sync_copy(idx_hbm.at[pl.ds(base + _CH, _CH)], idx_b)
    cp0 = pltpu.async_copy(cb_hbm.at[idx_a], rows_a, sem)
    cp1 = pltpu.async_copy(cb_hbm.at[idx_b], rows_b, sem)
    cp0.wait()
    cp1.wait()
    pltpu.sync_copy(rows_a, out_hbm.at[pl.ds(base, _CH)])
    pltpu.sync_copy(rows_b, out_hbm.at[pl.ds(base + _CH, _CH)])


def kernel(z, codebook):
    z_perm = jnp.transpose(z, (0, 2, 3, 1))
    z_flat = z_perm.reshape(-1, _DIM)
    idx = _tc_argmin(z_flat, codebook)
    z_q = _sc_gather(codebook, idx)
    return z_q.reshape(z_perm.shape), idx


# speedup vs baseline: 1.2271x; 1.2271x over previous
import functools

import jax
import jax.numpy as jnp
from jax import lax
from jax.experimental import pallas as pl
from jax.experimental.pallas import tpu as pltpu
from jax.experimental.pallas import tpu_sc as plsc

_DIM = 256
_NE = 8192
_NTOK = 8192
_BLK = 512


def _argmin_body(zb_ref, cb_ref, idx_ref):
    zb = zb_ref[...]
    cb = cb_ref[...]
    z2 = jnp.sum(zb * zb, axis=1, keepdims=True)
    e2 = jnp.sum(cb * cb, axis=1)
    prod = lax.dot_general(zb, cb, (((1,), (1,)), ((), ())),
                           preferred_element_type=jnp.float32)
    d = (z2 + e2) - 2.0 * prod
    m = jnp.min(d, axis=1, keepdims=True)
    ids = lax.broadcasted_iota(jnp.int32, d.shape, 1)
    idx_ref[...] = jnp.min(jnp.where(d == m, ids, jnp.int32(_NE)), axis=1)


def _build_argmin(interpret: bool = False):
    return pl.pallas_call(
        _argmin_body,
        grid=(_NTOK // _BLK,),
        in_specs=[
            pl.BlockSpec((_BLK, _DIM), lambda i: (i, 0)),
            pl.BlockSpec((_NE, _DIM), lambda i: (0, 0)),
        ],
        out_specs=pl.BlockSpec((_BLK,), lambda i: (i,)),
        out_shape=jax.ShapeDtypeStruct((_NTOK,), jnp.int32),
        compiler_params=pltpu.CompilerParams(
            dimension_semantics=("arbitrary",)),
        interpret=interpret,
    )


_tc_argmin = _build_argmin()

_NC = 2
_NS = 16
_NW = _NC * _NS
_BPW = _NTOK // _NW
_CH = 128

@functools.cache
def _build_sc_gather():
    mesh = plsc.VectorSubcoreMesh(core_axis_name="c", subcore_axis_name="s",
                                  num_cores=_NC, num_subcores=_NS)

    @functools.partial(
        pl.kernel,
        out_type=jax.ShapeDtypeStruct((_NTOK, _DIM), jnp.float32),
        mesh=mesh,
        scratch_types=[
            pltpu.VMEM((_CH,), jnp.int32),
            pltpu.VMEM((_CH,), jnp.int32),
            pltpu.VMEM((_CH, _DIM), jnp.float32),
            pltpu.VMEM((_CH, _DIM), jnp.float32),
            pltpu.SemaphoreType.DMA,
        ],
    )
    def _sc_gather(cb_hbm, idx_hbm, out_hbm, idx_a, idx_b, rows_a, rows_b, sem):
        wid = lax.axis_index("s") * _NC + lax.axis_index("c")
        base = wid * _BPW
        pltpu.sync_copy(idx_hbm.at[pl.ds(base, _CH)], idx_a)
        pltpu.sync_copy(idx_hbm.at[pl.ds(base + _CH, _CH)], idx_b)
        cp0 = pltpu.async_copy(cb_hbm.at[idx_a], rows_a, sem)
        cp1 = pltpu.async_copy(cb_hbm.at[idx_b], rows_b, sem)
        cp0.wait()
        cp1.wait()
        pltpu.sync_copy(rows_a, out_hbm.at[pl.ds(base, _CH)])
        pltpu.sync_copy(rows_b, out_hbm.at[pl.ds(base + _CH, _CH)])

    return _sc_gather


def kernel(z, codebook):
    z_perm = jnp.transpose(z, (0, 2, 3, 1))
    z_flat = z_perm.reshape(-1, _DIM)
    idx = _tc_argmin(z_flat, codebook)
    z_q = _build_sc_gather()(codebook, idx)
    return z_q.reshape(z_perm.shape), idx


# speedup vs baseline: 1.2625x; 1.0288x over previous
import functools

import jax
import jax.numpy as jnp
from jax import lax
from jax.experimental import pallas as pl
from jax.experimental.pallas import tpu as pltpu
from jax.experimental.pallas import tpu_sc as plsc

_DIM = 256
_NE = 8192
_NTOK = 8192
_BLK = 512


_W = 1024
_NCHUNK = _NE // _W


def _argmin_body(zb_ref, cb_ref, idx_ref, e2_ref):
    @pl.when(pl.program_id(0) == 0)
    def _():
        cb = cb_ref[...]
        e2_ref[...] = jnp.sum(cb * cb, axis=1)

    zb = zb_ref[...]
    z2 = jnp.sum(zb * zb, axis=1, keepdims=True)
    zb2 = zb + zb
    bv = None
    bi = None
    for j in range(_NCHUNK):
        cbj = cb_ref[pl.ds(j * _W, _W), :]
        pj = lax.dot_general(zb2, cbj, (((1,), (1,)), ((), ())),
                             preferred_element_type=jnp.float32)
        e2j = e2_ref[pl.ds(j * _W, _W)]
        d = (z2 + e2j) - pj
        if j == 0:
            bv = d
            bi = jnp.zeros(d.shape, jnp.int32)
        else:
            lt = d < bv
            bv = jnp.where(lt, d, bv)
            bi = jnp.where(lt, jnp.int32(j), bi)
    m = jnp.min(bv, axis=1, keepdims=True)
    col = lax.broadcasted_iota(jnp.int32, bv.shape, 1)
    gidx = bi * _W + col
    idx_ref[...] = jnp.min(jnp.where(bv == m, gidx, jnp.int32(_NE)), axis=1)


def _build_argmin(interpret: bool = False):
    return pl.pallas_call(
        _argmin_body,
        grid=(_NTOK // _BLK,),
        in_specs=[
            pl.BlockSpec((_BLK, _DIM), lambda i: (i, 0)),
            pl.BlockSpec((_NE, _DIM), lambda i: (0, 0)),
        ],
        out_specs=pl.BlockSpec((_BLK,), lambda i: (i,)),
        out_shape=jax.ShapeDtypeStruct((_NTOK,), jnp.int32),
        scratch_shapes=[pltpu.VMEM((_NE,), jnp.float32)],
        compiler_params=pltpu.CompilerParams(
            dimension_semantics=("arbitrary",)),
        interpret=interpret,
    )


_tc_argmin = _build_argmin()

_NC = 2
_NS = 16
_NW = _NC * _NS
_BPW = _NTOK // _NW
_CH = 128

@functools.cache
def _build_sc_gather():
    mesh = plsc.VectorSubcoreMesh(core_axis_name="c", subcore_axis_name="s",
                                  num_cores=_NC, num_subcores=_NS)

    @functools.partial(
        pl.kernel,
        out_type=jax.ShapeDtypeStruct((_NTOK, _DIM), jnp.float32),
        mesh=mesh,
        scratch_types=[
            pltpu.VMEM((_CH,), jnp.int32),
            pltpu.VMEM((_CH,), jnp.int32),
            pltpu.VMEM((_CH, _DIM), jnp.float32),
            pltpu.VMEM((_CH, _DIM), jnp.float32),
            pltpu.SemaphoreType.DMA,
        ],
    )
    def _sc_gather(cb_hbm, idx_hbm, out_hbm, idx_a, idx_b, rows_a, rows_b, sem):
        wid = lax.axis_index("s") * _NC + lax.axis_index("c")
        base = wid * _BPW
        pltpu.sync_copy(idx_hbm.at[pl.ds(base, _CH)], idx_a)
        pltpu.sync_copy(idx_hbm.at[pl.ds(base + _CH, _CH)], idx_b)
        cp0 = pltpu.async_copy(cb_hbm.at[idx_a], rows_a, sem)
        cp1 = pltpu.async_copy(cb_hbm.at[idx_b], rows_b, sem)
        cp0.wait()
        cp1.wait()
        pltpu.sync_copy(rows_a, out_hbm.at[pl.ds(base, _CH)])
        pltpu.sync_copy(rows_b, out_hbm.at[pl.ds(base + _CH, _CH)])

    return _sc_gather


def kernel(z, codebook):
    z_perm = jnp.transpose(z, (0, 2, 3, 1))
    z_flat = z_perm.reshape(-1, _DIM)
    idx = _tc_argmin(z_flat, codebook)
    z_q = _build_sc_gather()(codebook, idx)
    return z_q.reshape(z_perm.shape), idx
